# baseline (speedup 1.0000x reference)
import jax
import jax.numpy as jnp
from jax.experimental import pallas as pl
from jax.experimental.pallas import tpu as pltpu

_B, _H, _L, _D, _S = 8, 16, 2048, 128, 16
_NCHUNK = 8


def _body(pos_ref, kc, vc, kv, vv, ko, vo, valbuf, sem_val, sem_big, sem_sc):
    bh = _B * _H
    rows = _L // _NCHUNK
    cp_kv = pltpu.make_async_copy(kv, valbuf.at[0], sem_val)
    cp_vv = pltpu.make_async_copy(vv, valbuf.at[1], sem_val)
    cp_kv.start()
    cp_vv.start()
    big = []
    for c in range(_NCHUNK):
        sl = pl.ds(c * rows, rows)
        ck = pltpu.make_async_copy(kc.at[:, sl, :], ko.at[:, sl, :], sem_big)
        cv = pltpu.make_async_copy(vc.at[:, sl, :], vo.at[:, sl, :], sem_big)
        ck.start()
        cv.start()
        big.append(ck)
        big.append(cv)
    cp_kv.wait()
    cp_vv.wait()
    for cp in big:
        cp.wait()
    scat = []
    for i in range(_S):
        p = pos_ref[i]
        sk = pltpu.make_async_copy(
            valbuf.at[0, :, pl.ds(i, 1), :], ko.at[:, pl.ds(p, 1), :], sem_sc)
        sv = pltpu.make_async_copy(
            valbuf.at[1, :, pl.ds(i, 1), :], vo.at[:, pl.ds(p, 1), :], sem_sc)
        sk.start()
        sv.start()
        scat.append(sk)
        scat.append(sv)
    for cp in scat:
        cp.wait()


def kernel(input_pos, k_val, v_val, k_cache, v_cache):
    bh = _B * _H
    kc = k_cache.reshape(bh, _L, _D)
    vc = v_cache.reshape(bh, _L, _D)
    kv = k_val.reshape(bh, _S, _D)
    vv = v_val.reshape(bh, _S, _D)

    any_spec = pl.BlockSpec(memory_space=pl.ANY)
    ko, vo = pl.pallas_call(
        _body,
        in_specs=[
            pl.BlockSpec(memory_space=pltpu.SMEM),
            any_spec, any_spec, any_spec, any_spec,
        ],
        out_specs=[any_spec, any_spec],
        out_shape=[
            jax.ShapeDtypeStruct((bh, _L, _D), jnp.float32),
            jax.ShapeDtypeStruct((bh, _L, _D), jnp.float32),
        ],
        scratch_shapes=[
            pltpu.VMEM((2, bh, _S, _D), jnp.float32),
            pltpu.SemaphoreType.DMA,
            pltpu.SemaphoreType.DMA,
            pltpu.SemaphoreType.DMA,
        ],
    )(input_pos, kc, vc, kv, vv)
    return (ko.reshape(_B, _H, _L, _D), vo.reshape(_B, _H, _L, _D))


# speedup vs baseline: 1.0002x; 1.0002x over previous
import jax
import jax.numpy as jnp
from jax.experimental import pallas as pl
from jax.experimental.pallas import tpu as pltpu

_B, _H, _L, _D, _S = 8, 16, 2048, 128, 16
_NCHUNK = 8


def _body(pos_ref, kc, vc, kv, vv, ko, vo, valbuf, sem_val, sem_big, sem_sc):
    bh = _B * _H
    rows = _L // _NCHUNK
    cp_kv = pltpu.make_async_copy(kv, valbuf.at[0], sem_val)
    cp_vv = pltpu.make_async_copy(vv, valbuf.at[1], sem_val)
    cp_kv.start()
    cp_vv.start()
    big = []
    nbh = bh // _NCHUNK
    for c in range(_NCHUNK):
        sl = pl.ds(c * nbh, nbh)
        ck = pltpu.make_async_copy(kc.at[sl], ko.at[sl], sem_big)
        cv = pltpu.make_async_copy(vc.at[sl], vo.at[sl], sem_big)
        ck.start()
        cv.start()
        big.append(ck)
        big.append(cv)
    cp_kv.wait()
    cp_vv.wait()
    for cp in big:
        cp.wait()
    scat = []
    for i in range(_S):
        p = pos_ref[i]
        sk = pltpu.make_async_copy(
            valbuf.at[0, :, pl.ds(i, 1), :], ko.at[:, pl.ds(p, 1), :], sem_sc)
        sv = pltpu.make_async_copy(
            valbuf.at[1, :, pl.ds(i, 1), :], vo.at[:, pl.ds(p, 1), :], sem_sc)
        sk.start()
        sv.start()
        scat.append(sk)
        scat.append(sv)
    for cp in scat:
        cp.wait()


def kernel(input_pos, k_val, v_val, k_cache, v_cache):
    bh = _B * _H
    kc = k_cache.reshape(bh, _L, _D)
    vc = v_cache.reshape(bh, _L, _D)
    kv = k_val.reshape(bh, _S, _D)
    vv = v_val.reshape(bh, _S, _D)

    any_spec = pl.BlockSpec(memory_space=pl.ANY)
    ko, vo = pl.pallas_call(
        _body,
        in_specs=[
            pl.BlockSpec(memory_space=pltpu.SMEM),
            any_spec, any_spec, any_spec, any_spec,
        ],
        out_specs=[any_spec, any_spec],
        out_shape=[
            jax.ShapeDtypeStruct((bh, _L, _D), jnp.float32),
            jax.ShapeDtypeStruct((bh, _L, _D), jnp.float32),
        ],
        scratch_shapes=[
            pltpu.VMEM((2, bh, _S, _D), jnp.float32),
            pltpu.SemaphoreType.DMA,
            pltpu.SemaphoreType.DMA,
            pltpu.SemaphoreType.DMA,
        ],
    )(input_pos, kc, vc, kv, vv)
    return (ko.reshape(_B, _H, _L, _D), vo.reshape(_B, _H, _L, _D))


# speedup vs baseline: 48.4566x; 48.4458x over previous
import jax
import jax.numpy as jnp
from jax.experimental import pallas as pl
from jax.experimental.pallas import tpu as pltpu

_B, _H, _L, _D, _S = 8, 16, 2048, 128, 16
_GB = 4


def _body(pos_ref, kc_ref, vc_ref, kv_ref, vv_ref, ko_ref, vo_ref):
    ko_ref[...] = kc_ref[...]
    vo_ref[...] = vc_ref[...]
    for i in range(_S):
        p = pos_ref[i]
        for j in range(_GB):
            ko_ref[j, pl.ds(p, 1), :] = kv_ref[j, pl.ds(i, 1), :]
            vo_ref[j, pl.ds(p, 1), :] = vv_ref[j, pl.ds(i, 1), :]


def kernel(input_pos, k_val, v_val, k_cache, v_cache):
    bh = _B * _H
    kc = k_cache.reshape(bh, _L, _D)
    vc = v_cache.reshape(bh, _L, _D)
    kv = k_val.reshape(bh, _S, _D)
    vv = v_val.reshape(bh, _S, _D)

    cache_spec = pl.BlockSpec((_GB, _L, _D), lambda i: (i, 0, 0))
    val_spec = pl.BlockSpec((_GB, _S, _D), lambda i: (i, 0, 0))
    ko, vo = pl.pallas_call(
        _body,
        grid=(bh // _GB,),
        in_specs=[
            pl.BlockSpec(memory_space=pltpu.SMEM),
            cache_spec,
            cache_spec,
            val_spec,
            val_spec,
        ],
        out_specs=[cache_spec, cache_spec],
        out_shape=[
            jax.ShapeDtypeStruct((bh, _L, _D), jnp.float32),
            jax.ShapeDtypeStruct((bh, _L, _D), jnp.float32),
        ],
        compiler_params=pltpu.CompilerParams(
            dimension_semantics=("arbitrary",),
        ),
    )(input_pos, kc, vc, kv, vv)
    return (ko.reshape(_B, _H, _L, _D), vo.reshape(_B, _H, _L, _D))
